# baseline (speedup 1.0000x reference)
import functools

import jax
import jax.numpy as jnp
from jax import lax
from jax.experimental import pallas as pl
from jax.experimental.pallas import tpu as pltpu
from jax.experimental.pallas import tpu_sc as plsc

D = 128
NUM_CORES = 2
NUM_SUBCORES = 16
NW = NUM_CORES * NUM_SUBCORES
NBUF = 4


@functools.lru_cache(maxsize=None)
def _make_kernel(b_total: int, f_total: int):
    b_per_w = b_total // NW
    mesh = plsc.VectorSubcoreMesh(
        core_axis_name="c",
        subcore_axis_name="s",
        num_cores=NUM_CORES,
        num_subcores=NUM_SUBCORES,
    )

    @functools.partial(
        pl.kernel,
        out_type=jax.ShapeDtypeStruct((f_total, b_total, D), jnp.float32),
        mesh=mesh,
        scratch_types=[
            pltpu.VMEM((f_total, b_per_w), jnp.int32),
            pltpu.VMEM((NBUF, b_per_w, D), jnp.float32),
            pltpu.SemaphoreType.DMA,
            pltpu.SemaphoreType.DMA,
        ],
    )
    def emb(xt_hbm, w_hbm, out_hbm, idx_v, bufs, gsem, ssem):
        wid = lax.axis_index("s") * NUM_CORES + lax.axis_index("c")
        b0 = wid * b_per_w
        for f in range(f_total):
            pltpu.sync_copy(xt_hbm.at[f, pl.ds(b0, b_per_w)], idx_v.at[f])

        def gather(f, buf):
            pltpu.async_copy(w_hbm.at[idx_v.at[f]], buf, gsem)

        def gather_wait(f, buf):
            pltpu.make_async_copy(w_hbm.at[idx_v.at[f]], buf, gsem).wait()

        def drain_one():
            pltpu.make_async_copy(
                bufs.at[0], out_hbm.at[0, pl.ds(0, b_per_w)], ssem
            ).wait()

        gather(0, bufs.at[0])
        gather(1, bufs.at[1])

        def body(f, carry):
            buf = bufs.at[lax.rem(f, NBUF)]
            gather_wait(f, buf)
            pltpu.async_copy(buf, out_hbm.at[f, pl.ds(b0, b_per_w)], ssem)

            @pl.when(f >= 2)
            def _drain():
                drain_one()

            @pl.when(f + 2 < f_total)
            def _next():
                gather(f + 2, bufs.at[lax.rem(f + 2, NBUF)])

            return carry

        lax.fori_loop(0, f_total, body, 0)

        drain_one()
        drain_one()

    return emb


def kernel(x, weight):
    b, f = x.shape
    xt = jnp.transpose(x).astype(jnp.int32)
    out = _make_kernel(b, f)(xt, weight)
    return jnp.transpose(out, (1, 0, 2))


# speedup vs baseline: 1.1916x; 1.1916x over previous
import functools

import jax
import jax.numpy as jnp
from jax import lax
from jax.experimental import pallas as pl
from jax.experimental.pallas import tpu as pltpu
from jax.experimental.pallas import tpu_sc as plsc

D = 128
NUM_CORES = 2
NUM_SUBCORES = 16
NW = NUM_CORES * NUM_SUBCORES
NBUF = 6
PRIME = 3


@functools.lru_cache(maxsize=None)
def _make_kernel(b_total: int, f_total: int):
    b_per_w = b_total // NW
    mesh = plsc.VectorSubcoreMesh(
        core_axis_name="c",
        subcore_axis_name="s",
        num_cores=NUM_CORES,
        num_subcores=NUM_SUBCORES,
    )

    @functools.partial(
        pl.kernel,
        out_type=jax.ShapeDtypeStruct((f_total, b_total, D), jnp.float32),
        mesh=mesh,
        scratch_types=[
            pltpu.VMEM((f_total, b_per_w), jnp.int32),
            pltpu.VMEM((NBUF, b_per_w, D), jnp.float32),
            pltpu.SemaphoreType.DMA,
            pltpu.SemaphoreType.DMA,
            pltpu.SemaphoreType.DMA,
        ],
    )
    def emb(xt_hbm, w_hbm, out_hbm, idx_v, bufs, gsem, ssem, stsem):
        wid = lax.axis_index("s") * NUM_CORES + lax.axis_index("c")
        b0 = wid * b_per_w

        for f in range(f_total):
            pltpu.async_copy(xt_hbm.at[f, pl.ds(b0, b_per_w)], idx_v.at[f], stsem)
        for f in range(f_total):
            pltpu.make_async_copy(
                xt_hbm.at[0, pl.ds(b0, b_per_w)], idx_v.at[0], stsem
            ).wait()

        def gather(f, buf):
            pltpu.async_copy(w_hbm.at[idx_v.at[f]], buf, gsem)

        def gather_wait(f, buf):
            pltpu.make_async_copy(w_hbm.at[idx_v.at[f]], buf, gsem).wait()

        def drain_one():
            pltpu.make_async_copy(
                bufs.at[0], out_hbm.at[0, pl.ds(0, b_per_w)], ssem
            ).wait()

        for f in range(PRIME):
            gather(f, bufs.at[f])

        def body(f, carry):
            buf = bufs.at[lax.rem(f, NBUF)]
            gather_wait(f, buf)
            pltpu.async_copy(buf, out_hbm.at[f, pl.ds(b0, b_per_w)], ssem)

            @pl.when(f >= NBUF - PRIME)
            def _drain():
                drain_one()

            @pl.when(f + PRIME < f_total)
            def _next():
                fn = f + PRIME
                gather(fn, bufs.at[lax.rem(fn, NBUF)])

            return carry

        lax.fori_loop(0, f_total, body, 0)

        for _ in range(PRIME):
            drain_one()

    return emb


def kernel(x, weight):
    b, f = x.shape
    xt = jnp.transpose(x).astype(jnp.int32)
    out = _make_kernel(b, f)(xt, weight)
    return jnp.transpose(out, (1, 0, 2))
